# baseline (speedup 1.0000x reference)
import functools

import jax
import jax.numpy as jnp
from jax import lax
from jax.experimental import pallas as pl
from jax.experimental.pallas import tpu as pltpu
from jax.experimental.pallas import tpu_sc as plsc

N_STATES = 32 * 32 * 32 * 32
N_ACTIONS = 16
BATCH = 16384

NC = 2
NS = 16
L = 16
NW = NC * NS
B_PER_W = BATCH // NW
E_PER_W = B_PER_W * N_ACTIONS

_ACTION_OFF = [(a >> 3) * (8 * N_STATES) + (a & 7) * 128
               for a in range(N_ACTIONS)]
_SPAN = N_ACTIONS * N_STATES - max(_ACTION_OFF)

_mesh = plsc.VectorSubcoreMesh(core_axis_name="c", subcore_axis_name="s")


@functools.partial(
    pl.kernel,
    out_type=jax.ShapeDtypeStruct((N_ACTIONS, BATCH), jnp.float32),
    mesh=_mesh,
    scratch_types=[
        pltpu.VMEM((B_PER_W,), jnp.int32),
        pltpu.VMEM((E_PER_W,), jnp.float32),
        pltpu.SemaphoreType.DMA,
        pltpu.SemaphoreType.DMA,
    ],
)
def _qlookup(w_hbm, obs_hbm, out_hbm, sidx_v, rows_v, sem_lo, sem_hi):
    wid = lax.axis_index("s") * NC + lax.axis_index("c")
    base = wid * B_PER_W
    H = B_PER_W // 2

    obs_cp = [
        pltpu.async_copy(obs_hbm.at[pl.ds(base + h * H, H)],
                         sidx_v.at[pl.ds(h * H, H)],
                         sem_lo if h == 0 else sem_hi)
        for h in range(2)
    ]

    def build(j, carry):
        o = sidx_v[pl.ds(j * L, L)]
        sidx_v[pl.ds(j * L, L)] = ((o >> 7) << 10) + (o & 127)
        return carry

    copies = []
    for h in range(2):
        obs_cp[h].wait()
        lax.fori_loop(h * (H // L), (h + 1) * (H // L), build, 0)
        copies += [
            pltpu.async_copy(
                w_hbm.at[pl.ds(_ACTION_OFF[a], _SPAN)].at[
                    sidx_v.at[pl.ds(h * H, H)]],
                rows_v.at[pl.ds(a * B_PER_W + h * H, H)],
                sem_lo if h == 0 else sem_hi)
            for a in range(N_ACTIONS)
        ]

    for c in copies[:N_ACTIONS]:
        c.wait()
    for a in range(N_ACTIONS):
        pltpu.sync_copy(rows_v.at[pl.ds(a * B_PER_W, H)],
                        out_hbm.at[a, pl.ds(base, H)])
    for c in copies[N_ACTIONS:]:
        c.wait()
    for a in range(N_ACTIONS):
        pltpu.sync_copy(rows_v.at[pl.ds(a * B_PER_W + H, H)],
                        out_hbm.at[a, pl.ds(base + H, H)])


def kernel(v_obs, W):
    w_flat = (W.reshape(2, 8, 8192, 128)
               .transpose(0, 2, 1, 3)
               .reshape(N_ACTIONS * N_STATES))
    out = _qlookup(w_flat, v_obs.astype(jnp.int32))
    return out.T


# speedup vs baseline: 1.0387x; 1.0387x over previous
import functools

import jax
import jax.numpy as jnp
from jax import lax
from jax.experimental import pallas as pl
from jax.experimental.pallas import tpu as pltpu
from jax.experimental.pallas import tpu_sc as plsc

N_STATES = 32 * 32 * 32 * 32
N_ACTIONS = 16
BATCH = 16384

NC = 2
NS = 16
L = 16
NW = NC * NS
B_PER_W = BATCH // NW
E_PER_W = B_PER_W * N_ACTIONS

_ACTION_OFF = [(a >> 3) * (8 * N_STATES) + (a & 7) * 128
               for a in range(N_ACTIONS)]
_SPAN = N_ACTIONS * N_STATES - max(_ACTION_OFF)

_mesh = plsc.VectorSubcoreMesh(core_axis_name="c", subcore_axis_name="s")


@functools.partial(
    pl.kernel,
    out_type=jax.ShapeDtypeStruct((N_ACTIONS, BATCH), jnp.float32),
    mesh=_mesh,
    scratch_types=[
        pltpu.VMEM((B_PER_W,), jnp.int32),
        pltpu.VMEM((E_PER_W,), jnp.float32),
        pltpu.SemaphoreType.DMA,
        pltpu.SemaphoreType.DMA,
    ],
)
def _qlookup(w_hbm, obs_hbm, out_hbm, sidx_v, rows_v, sem_lo, sem_hi):
    wid = lax.axis_index("s") * NC + lax.axis_index("c")
    base = wid * B_PER_W
    pltpu.sync_copy(obs_hbm.at[pl.ds(base, B_PER_W)], sidx_v)

    def build(j, carry):
        o = sidx_v[pl.ds(j * L, L)]
        sidx_v[pl.ds(j * L, L)] = ((o >> 7) << 10) + (o & 127)
        return carry

    lax.fori_loop(0, B_PER_W // L, build, 0)

    half = N_ACTIONS // 2
    copies = [
        pltpu.async_copy(
            w_hbm.at[pl.ds(_ACTION_OFF[a], _SPAN)].at[sidx_v],
            rows_v.at[pl.ds(a * B_PER_W, B_PER_W)],
            sem_lo if a < half else sem_hi)
        for a in range(N_ACTIONS)
    ]
    for c in copies[:half]:
        c.wait()
    for a in range(half):
        pltpu.sync_copy(rows_v.at[pl.ds(a * B_PER_W, B_PER_W)],
                        out_hbm.at[a, pl.ds(base, B_PER_W)])
    for c in copies[half:]:
        c.wait()
    for a in range(half, N_ACTIONS):
        pltpu.sync_copy(rows_v.at[pl.ds(a * B_PER_W, B_PER_W)],
                        out_hbm.at[a, pl.ds(base, B_PER_W)])


def kernel(v_obs, W):
    w_flat = (W.reshape(2, 8, 8192, 128)
               .transpose(0, 2, 1, 3)
               .reshape(N_ACTIONS * N_STATES))
    out = _qlookup(w_flat, v_obs.astype(jnp.int32))
    return out.T
